# Initial kernel scaffold; baseline (speedup 1.0000x reference)
import jax
import jax.numpy as jnp
from jax.experimental import pallas as pl


def kernel(edge_vec, edge_index, pos, edge_w):
    raise NotImplementedError("write your pallas kernel here")


# speedup vs baseline: 3.2356x; 3.2356x over previous
import functools

import jax
import jax.numpy as jnp
from jax import lax
from jax.experimental import pallas as pl
from jax.experimental.pallas import tpu as pltpu
from jax.experimental.pallas import tpu_sc as plsc

NC = 2
NS = 16
NW = NC * NS
LANES = 16


def _sc_scatter(n_edges, n_pad, block):
    n_blocks_g = n_edges // block
    n_iters = (n_blocks_g + NW - 1) // NW
    wpt = n_pad * 4 // NS
    mesh = plsc.VectorSubcoreMesh(core_axis_name="c", subcore_axis_name="s")

    @functools.partial(
        pl.kernel,
        out_type=(
            jax.ShapeDtypeStruct((NC, n_pad * 4), jnp.float32),
            jax.ShapeDtypeStruct((NC, n_pad * 4), jnp.float32),
        ),
        mesh=mesh,
        compiler_params=pltpu.CompilerParams(
            needs_layout_passes=False, use_tc_tiling_on_sc=False),
        scratch_types=[
            pltpu.VMEM((3 * block,), jnp.float32),
            pltpu.VMEM((block,), jnp.float32),
            pltpu.VMEM((block,), jnp.int32),
            pltpu.VMEM((block,), jnp.int32),
            pltpu.VMEM((4 * block,), jnp.float32),
            pltpu.VMEM((4 * block,), jnp.int32),
            pltpu.VMEM((4 * block,), jnp.int32),
            pltpu.VMEM_SHARED((n_pad * 4,), jnp.float32),
            pltpu.VMEM_SHARED((n_pad * 4,), jnp.float32),
        ],
    )
    def k(ev_hbm, w_hbm, src_hbm, dst_hbm, zero_hbm, outp_hbm, outn_hbm,
          ev_v, w_v, si_v, di_v, f4_v, siw_v, diw_v, acc_p, acc_n):
        lane = lax.broadcasted_iota(jnp.int32, (LANES,), 0)
        rg = lane >> 2
        col = lane & 3
        ev_idx0 = rg * 3 + jnp.minimum(col, 2)
        colmask = col < 3

        cid = lax.axis_index("c")
        sid = lax.axis_index("s")
        wid = cid * NS + sid
        w0 = sid * wpt

        pltpu.sync_copy(zero_hbm, acc_p.at[pl.ds(w0, wpt)])
        pltpu.sync_copy(zero_hbm, acc_n.at[pl.ds(w0, wpt)])
        plsc.subcore_barrier()

        def body(g):
            base = g * block
            pltpu.sync_copy(ev_hbm.at[pl.ds(3 * base, 3 * block)], ev_v)
            pltpu.sync_copy(w_hbm.at[pl.ds(base, block)], w_v)
            pltpu.sync_copy(src_hbm.at[pl.ds(base, block)], si_v)
            pltpu.sync_copy(dst_hbm.at[pl.ds(base, block)], di_v)

            def grp(j, _):
                eb = j * 4
                q = j * LANES
                evg = plsc.load_gather(ev_v, [ev_idx0 + 3 * eb])
                wg = plsc.load_gather(w_v, [rg + eb])
                sg = plsc.load_gather(si_v, [rg + eb])
                dg = plsc.load_gather(di_v, [rg + eb])
                val = jnp.where(colmask, evg * (wg + wg), 0.0)
                f4_v[pl.ds(q, LANES)] = val
                siw_v[pl.ds(q, LANES)] = sg * 4 + col
                diw_v[pl.ds(q, LANES)] = dg * 4 + col
                return 0

            lax.fori_loop(0, block // 4, grp, 0)
            pltpu.sync_copy(f4_v, acc_p.at[siw_v], add=True)
            pltpu.sync_copy(f4_v, acc_n.at[diw_v], add=True)

        def blk(b, _):
            g = b * NW + wid
            @pl.when(g < n_blocks_g)
            def _():
                body(g)
            return 0

        lax.fori_loop(0, n_iters, blk, 0)
        plsc.subcore_barrier()

        pltpu.sync_copy(acc_p.at[pl.ds(w0, wpt)],
                        outp_hbm.at[cid, pl.ds(w0, wpt)])
        pltpu.sync_copy(acc_n.at[pl.ds(w0, wpt)],
                        outn_hbm.at[cid, pl.ds(w0, wpt)])

    return k


def _combine_body(p_ref, n_ref, o_ref):
    o_ref[...] = (p_ref[0] + p_ref[1]) - (n_ref[0] + n_ref[1])


def kernel(edge_vec, edge_index, pos, edge_w):
    n_edges = edge_vec.shape[0]
    n_nodes = pos.shape[0]
    block = 2048
    n_pad = ((n_nodes + 127) // 128) * 128
    assert n_edges % block == 0

    ev_flat = edge_vec.reshape(-1)
    src = edge_index[0]
    dst = edge_index[1]
    zero = jnp.zeros((n_pad * 4 // NS,), jnp.float32)

    part_p, part_n = _sc_scatter(n_edges, n_pad, block)(
        ev_flat, edge_w, src, dst, zero)

    rows = n_pad * 4 // 128
    blk_rows = 184
    p2 = part_p.reshape(NC, rows, 128)
    n2 = part_n.reshape(NC, rows, 128)
    out = pl.pallas_call(
        _combine_body,
        grid=(rows // blk_rows,),
        in_specs=[
            pl.BlockSpec((NC, blk_rows, 128), lambda i: (0, i, 0)),
            pl.BlockSpec((NC, blk_rows, 128), lambda i: (0, i, 0)),
        ],
        out_specs=pl.BlockSpec((blk_rows, 128), lambda i: (i, 0)),
        out_shape=jax.ShapeDtypeStruct((rows, 128), jnp.float32),
    )(p2, n2)
    return out.reshape(n_pad, 4)[:n_nodes, :3]


# speedup vs baseline: 3.2391x; 1.0011x over previous
import functools

import jax
import jax.numpy as jnp
from jax import lax
from jax.experimental import pallas as pl
from jax.experimental.pallas import tpu as pltpu
from jax.experimental.pallas import tpu_sc as plsc

NC = 2
NS = 16
NW = NC * NS
LANES = 16


def _sc_scatter(n_edges, n_pad, block):
    n_blocks_g = n_edges // block
    n_iters = (n_blocks_g + NW - 1) // NW
    wpt = n_pad * 4 // NS
    mesh = plsc.VectorSubcoreMesh(core_axis_name="c", subcore_axis_name="s")

    @functools.partial(
        pl.kernel,
        out_type=(
            jax.ShapeDtypeStruct((NC, n_pad * 4), jnp.float32),
            jax.ShapeDtypeStruct((NC, n_pad * 4), jnp.float32),
        ),
        mesh=mesh,
        compiler_params=pltpu.CompilerParams(
            needs_layout_passes=False, use_tc_tiling_on_sc=False),
        scratch_types=[
            pltpu.VMEM((3 * block,), jnp.float32),
            pltpu.VMEM((block,), jnp.float32),
            pltpu.VMEM((block,), jnp.int32),
            pltpu.VMEM((block,), jnp.int32),
            pltpu.VMEM((4 * block,), jnp.float32),
            pltpu.VMEM((4 * block,), jnp.int32),
            pltpu.VMEM((4 * block,), jnp.int32),
            pltpu.VMEM_SHARED((n_pad * 4,), jnp.float32),
            pltpu.VMEM_SHARED((n_pad * 4,), jnp.float32),
        ],
    )
    def k(ev_hbm, w_hbm, ei_hbm, zero_hbm, outp_hbm, outn_hbm,
          ev_v, w_v, si_v, di_v, f4_v, siw_v, diw_v, acc_p, acc_n):
        lane = lax.broadcasted_iota(jnp.int32, (LANES,), 0)
        rg = lane >> 2
        col = lane & 3
        ev_idx0 = rg * 3 + jnp.minimum(col, 2)
        colmask = col < 3

        cid = lax.axis_index("c")
        sid = lax.axis_index("s")
        wid = cid * NS + sid
        w0 = sid * wpt

        pltpu.sync_copy(zero_hbm, acc_p.at[pl.ds(w0, wpt)])
        pltpu.sync_copy(zero_hbm, acc_n.at[pl.ds(w0, wpt)])
        plsc.subcore_barrier()

        def body(g):
            base = g * block
            pltpu.sync_copy(ev_hbm.at[pl.ds(3 * base, 3 * block)], ev_v)
            pltpu.sync_copy(w_hbm.at[pl.ds(base, block)], w_v)
            pltpu.sync_copy(ei_hbm.at[0, pl.ds(base, block)], si_v)
            pltpu.sync_copy(ei_hbm.at[1, pl.ds(base, block)], di_v)

            def grp(j, _):
                eb = j * 4
                q = j * LANES
                evg = plsc.load_gather(ev_v, [ev_idx0 + 3 * eb])
                wg = plsc.load_gather(w_v, [rg + eb])
                sg = plsc.load_gather(si_v, [rg + eb])
                dg = plsc.load_gather(di_v, [rg + eb])
                val = jnp.where(colmask, evg * (wg + wg), 0.0)
                f4_v[pl.ds(q, LANES)] = val
                siw_v[pl.ds(q, LANES)] = sg * 4 + col
                diw_v[pl.ds(q, LANES)] = dg * 4 + col
                return 0

            lax.fori_loop(0, block // 4, grp, 0)
            pltpu.sync_copy(f4_v, acc_p.at[siw_v], add=True)
            pltpu.sync_copy(f4_v, acc_n.at[diw_v], add=True)

        def blk(b, _):
            g = b * NW + wid
            @pl.when(g < n_blocks_g)
            def _():
                body(g)
            return 0

        lax.fori_loop(0, n_iters, blk, 0)
        plsc.subcore_barrier()

        pltpu.sync_copy(acc_p.at[pl.ds(w0, wpt)],
                        outp_hbm.at[cid, pl.ds(w0, wpt)])
        pltpu.sync_copy(acc_n.at[pl.ds(w0, wpt)],
                        outn_hbm.at[cid, pl.ds(w0, wpt)])

    return k


def _combine_body(p_ref, n_ref, o_ref):
    o_ref[...] = (p_ref[0] + p_ref[1]) - (n_ref[0] + n_ref[1])


def kernel(edge_vec, edge_index, pos, edge_w):
    n_edges = edge_vec.shape[0]
    n_nodes = pos.shape[0]
    block = 2048
    n_pad = ((n_nodes + 127) // 128) * 128
    assert n_edges % block == 0

    ev_flat = edge_vec.reshape(-1)
    zero = jnp.zeros((n_pad * 4 // NS,), jnp.float32)

    part_p, part_n = _sc_scatter(n_edges, n_pad, block)(
        ev_flat, edge_w, edge_index, zero)

    rows = n_pad * 4 // 128
    blk_rows = 184
    p2 = part_p.reshape(NC, rows, 128)
    n2 = part_n.reshape(NC, rows, 128)
    out = pl.pallas_call(
        _combine_body,
        grid=(rows // blk_rows,),
        in_specs=[
            pl.BlockSpec((NC, blk_rows, 128), lambda i: (0, i, 0)),
            pl.BlockSpec((NC, blk_rows, 128), lambda i: (0, i, 0)),
        ],
        out_specs=pl.BlockSpec((blk_rows, 128), lambda i: (i, 0)),
        out_shape=jax.ShapeDtypeStruct((rows, 128), jnp.float32),
    )(p2, n2)
    return out.reshape(n_pad, 4)[:n_nodes, :3]
